# baseline (speedup 1.0000x reference)
import jax
import jax.numpy as jnp
from jax.experimental import pallas as pl

B = 8
H = 1024
S = 64
EPS = 1e-12


def _embed_ln_kernel(x_ref, bias_ref, gamma_ref, beta_ref, out_ref):
    x = x_ref[0]
    xt = x.T
    e = xt + bias_ref[...]
    mean = jnp.mean(e, axis=1, keepdims=True)
    ec = e - mean
    var = jnp.mean(ec * ec, axis=1, keepdims=True)
    inv = jax.lax.rsqrt(var + EPS)
    out_ref[0] = ec * inv * gamma_ref[...] + beta_ref[...]


def _bias_kernel(pos_ref, tok_ref, bias_ref):
    bias_ref[...] = pos_ref[...] + tok_ref[1, :][None, :]


def kernel(input_ids, pos_table, tok_table, ln_gamma, ln_beta):
    bias = pl.pallas_call(
        _bias_kernel,
        out_shape=jax.ShapeDtypeStruct((S, H), jnp.float32),
    )(pos_table, tok_table)

    gamma2 = ln_gamma.reshape(1, H)
    beta2 = ln_beta.reshape(1, H)

    out = pl.pallas_call(
        _embed_ln_kernel,
        grid=(B,),
        in_specs=[
            pl.BlockSpec((1, H, S), lambda b: (b, 0, 0)),
            pl.BlockSpec((S, H), lambda b: (0, 0)),
            pl.BlockSpec((1, H), lambda b: (0, 0)),
            pl.BlockSpec((1, H), lambda b: (0, 0)),
        ],
        out_specs=pl.BlockSpec((1, S, H), lambda b: (b, 0, 0)),
        out_shape=jax.ShapeDtypeStruct((B, S, H), jnp.float32),
    )(input_ids, bias, gamma2, beta2)
    return out


# speedup vs baseline: 1.0259x; 1.0259x over previous
import jax
import jax.numpy as jnp
from jax.experimental import pallas as pl
from jax.experimental.pallas import tpu as pltpu

B = 8
H = 1024
S = 64
EPS = 1e-12


def _embed_ln_kernel(x_ref, pos_ref, tok_ref, gamma_ref, beta_ref, out_ref):
    x = x_ref[0]
    xt = x.T
    e = xt + (pos_ref[...] + tok_ref[...])
    mean = jnp.mean(e, axis=1, keepdims=True)
    ec = e - mean
    var = jnp.mean(ec * ec, axis=1, keepdims=True)
    inv = jax.lax.rsqrt(var + EPS)
    out_ref[0] = ec * inv * gamma_ref[...] + beta_ref[...]


def kernel(input_ids, pos_table, tok_table, ln_gamma, ln_beta):
    gamma2 = ln_gamma.reshape(1, H)
    beta2 = ln_beta.reshape(1, H)
    tok_row = tok_table[1:2, :]

    out = pl.pallas_call(
        _embed_ln_kernel,
        grid=(B,),
        in_specs=[
            pl.BlockSpec((1, H, S), lambda b: (b, 0, 0)),
            pl.BlockSpec((S, H), lambda b: (0, 0)),
            pl.BlockSpec((1, H), lambda b: (0, 0)),
            pl.BlockSpec((1, H), lambda b: (0, 0)),
            pl.BlockSpec((1, H), lambda b: (0, 0)),
        ],
        out_specs=pl.BlockSpec((1, S, H), lambda b: (b, 0, 0)),
        out_shape=jax.ShapeDtypeStruct((B, S, H), jnp.float32),
        compiler_params=pltpu.CompilerParams(
            dimension_semantics=("arbitrary",),
        ),
    )(input_ids, pos_table, tok_row, gamma2, beta2)
    return out
